# Initial kernel scaffold; baseline (speedup 1.0000x reference)
import jax
import jax.numpy as jnp
from jax.experimental import pallas as pl


def kernel(x, vec, senders, receivers, interaction_matrices, ln_scale, ln_bias, p_W1, p_b1, p_W2, p_b2, q_W1, q_b1, q_W2, q_b2):
    raise NotImplementedError("write your pallas kernel here")


# speedup vs baseline: 14.2644x; 14.2644x over previous
import jax
import jax.numpy as jnp
from jax import lax
from jax.experimental import pallas as pl
from jax.experimental.pallas import tpu as pltpu
from jax.experimental.pallas import tpu_sc as plsc

_EPS = 1e-05

_NC = 2
_NS = 16
_NQ = 4
_K = 80


def _front_body(x_ref, vec_ref, lns, lnb, pw1, pb1, pw2, pb2,
                qw1, qb1, qw2, qb2, p_out, u4_out):
    x = x_ref[...]
    mean = jnp.mean(x, axis=-1, keepdims=True)
    cx = x - mean
    var = jnp.mean(cx * cx, axis=-1, keepdims=True)
    xn = cx * lax.rsqrt(var + _EPS) * lns[...] + lnb[...]

    def _mlp(w1, b1, w2, b2):
        h = jnp.dot(xn, w1[...], preferred_element_type=jnp.float32) + b1[...]
        h = h * jax.nn.sigmoid(h)
        return jnp.dot(h, w2[...], preferred_element_type=jnp.float32) + b2[...]

    p = _mlp(pw1, pb1, pw2, pb2)
    q = _mlp(qw1, qb1, qw2, qb2)
    u = q[:, None, :] * vec_ref[...]
    qw = u.shape[-1] // _NQ
    hq = [
        jnp.concatenate([u[:, 0, t * qw:(t + 1) * qw],
                         u[:, 1, t * qw:(t + 1) * qw],
                         u[:, 2, t * qw:(t + 1) * qw]], axis=-1)[:, None, :]
        for t in range(_NQ)
    ]
    bn = x.shape[0]
    u4 = jnp.concatenate(hq, axis=1).reshape(_NQ * bn, 3 * qw)
    p_out[...] = p
    u4_out[...] = u4


def _final_body(acc_ref, p_ref, out_ref):
    parts = [acc_ref[t] for t in range(_NQ)]
    out_ref[...] = jnp.concatenate(parts, axis=-1) * p_ref[...][:, None, :]


def _make_edge_body(n_nodes, n_edges, row_w):
    edges_per_tile = n_edges // _NS
    chunks = edges_per_tile // _K
    rows_per_tile = n_nodes // _NS
    zfull = rows_per_tile // _K
    zrem = rows_per_tile - zfull * _K
    nvr = row_w // 16

    def body(u4, send, recv, amat, acc_out, idx_v, recv_v, a_v, rows_v,
             zero_v, sem, acc_sh):
        core = lax.axis_index("c")
        sid = lax.axis_index("s")
        zero16 = jnp.zeros((16,), jnp.float32)

        def zrow(r, _):
            for t in range(nvr):
                zero_v[r, pl.ds(t * 16, 16)] = zero16
            return 0
        lax.fori_loop(0, _K, zrow, 0)

        row0 = sid * rows_per_tile
        e_base = sid * edges_per_tile

        for p in range(_NQ // _NC):
            for z in range(zfull):
                pltpu.sync_copy(zero_v, acc_sh.at[pl.ds(row0 + z * _K, _K)])
            if zrem:
                pltpu.sync_copy(zero_v.at[pl.ds(0, zrem)],
                                acc_sh.at[pl.ds(row0 + zfull * _K, zrem)])
            plsc.subcore_barrier()

            def chunk(ci, _):
                eoff = e_base + ci * _K
                pltpu.sync_copy(send.at[pl.ds(eoff, _K)], idx_v)

                def fix(t, _):
                    s = idx_v[pl.ds(t * 16, 16)]
                    idx_v[pl.ds(t * 16, 16)] = s * _NQ + (core + _NC * p)
                    return 0
                lax.fori_loop(0, _K // 16, fix, 0)

                pltpu.sync_copy(recv.at[pl.ds(eoff, _K)], recv_v)
                pltpu.sync_copy(amat.at[pl.ds(eoff, _K)], a_v)
                pltpu.async_copy(u4.at[idx_v], rows_v, sem).wait()

                def edge(e, _):
                    av = a_v[e, pl.ds(0, 16)]
                    a = [av[k] for k in range(9)]
                    u = [rows_v[e, pl.ds(t * 16, 16)] for t in range(nvr)]
                    for i in range(3):
                        for t in range(2):
                            o = (a[3 * i] * u[t] + a[3 * i + 1] * u[2 + t]
                                 + a[3 * i + 2] * u[4 + t])
                            rows_v[e, pl.ds((i * 2 + t) * 16, 16)] = o
                    return 0
                lax.fori_loop(0, _K, edge, 0)

                pltpu.sync_copy(rows_v, acc_sh.at[recv_v], add=True)
                return 0
            lax.fori_loop(0, chunks, chunk, 0)
            plsc.subcore_barrier()

            pltpu.sync_copy(acc_sh.at[pl.ds(row0, rows_per_tile)],
                            acc_out.at[core + _NC * p,
                                       pl.ds(row0, rows_per_tile)])

    return body


def kernel(x, vec, senders, receivers, interaction_matrices,
           ln_scale, ln_bias, p_W1, p_b1, p_W2, p_b2,
           q_W1, q_b1, q_W2, q_b2):
    n, c = x.shape
    e = senders.shape[0]
    qw = c // _NQ
    row_w = 3 * qw
    bn = 1000
    grid = n // bn

    wspec = lambda shape: pl.BlockSpec(shape, lambda i: (0,) * len(shape))
    p_arr, u4 = pl.pallas_call(
        _front_body,
        grid=(grid,),
        in_specs=[
            pl.BlockSpec((bn, c), lambda i: (i, 0)),
            pl.BlockSpec((bn, 3, c), lambda i: (i, 0, 0)),
            wspec((1, c)), wspec((1, c)),
            wspec((c, 2 * c)), wspec((1, 2 * c)), wspec((2 * c, c)), wspec((1, c)),
            wspec((c, 2 * c)), wspec((1, 2 * c)), wspec((2 * c, c)), wspec((1, c)),
        ],
        out_specs=[
            pl.BlockSpec((bn, c), lambda i: (i, 0)),
            pl.BlockSpec((_NQ * bn, row_w), lambda i: (i, 0)),
        ],
        out_shape=[
            jax.ShapeDtypeStruct((n, c), jnp.float32),
            jax.ShapeDtypeStruct((_NQ * n, row_w), jnp.float32),
        ],
    )(x, vec, ln_scale.reshape(1, c), ln_bias.reshape(1, c),
      p_W1, p_b1.reshape(1, 2 * c), p_W2, p_b2.reshape(1, c),
      q_W1, q_b1.reshape(1, 2 * c), q_W2, q_b2.reshape(1, c))

    amat = interaction_matrices.reshape(e, 9)
    amat = jnp.concatenate([amat, jnp.zeros((e, 7), jnp.float32)], axis=1)
    edge_fn = pl.kernel(
        _make_edge_body(n, e, row_w),
        out_type=jax.ShapeDtypeStruct((_NQ, n, row_w), jnp.float32),
        mesh=plsc.VectorSubcoreMesh(core_axis_name="c", subcore_axis_name="s",
                                    num_cores=_NC, num_subcores=_NS),
        compiler_params=pltpu.CompilerParams(use_tc_tiling_on_sc=False),
        scratch_types=[
            pltpu.VMEM((_K,), jnp.int32),
            pltpu.VMEM((_K,), jnp.int32),
            pltpu.VMEM((_K, 16), jnp.float32),
            pltpu.VMEM((_K, row_w), jnp.float32),
            pltpu.VMEM((_K, row_w), jnp.float32),
            pltpu.SemaphoreType.DMA,
            pltpu.VMEM_SHARED((n, row_w), jnp.float32),
        ],
    )
    acc = edge_fn(u4, senders, receivers, amat)
    acc4 = acc.reshape(_NQ, n, 3, qw)

    dvec = pl.pallas_call(
        _final_body,
        grid=(grid,),
        in_specs=[
            pl.BlockSpec((_NQ, bn, 3, qw), lambda i: (0, i, 0, 0)),
            pl.BlockSpec((bn, c), lambda i: (i, 0)),
        ],
        out_specs=pl.BlockSpec((bn, 3, c), lambda i: (i, 0, 0)),
        out_shape=jax.ShapeDtypeStruct((n, 3, c), jnp.float32),
    )(acc4, p_arr)
    return dvec
